# baseline (speedup 1.0000x reference)
import functools

import jax
import jax.numpy as jnp
from jax import lax
from jax.experimental import pallas as pl
from jax.experimental.pallas import tpu as pltpu
from jax.experimental.pallas import tpu_sc as plsc

VOCAB = 30522
HID = 768
PVOCAB = 100
PLEN = 20
B = 32
S = 512
EPS = 1e-12

NW = 32
NCHB = 1
CB = B // NCHB
WPB = NW // CB
ROWS_PER_W = CB * S // NW
CHUNK = 64
NCHUNK = ROWS_PER_W // CHUNK
PPAD = 24


def _sc_gather_chunk(word_emb, prompt_emb, wids_c, pids_c):
    mesh = plsc.VectorSubcoreMesh(core_axis_name="c", subcore_axis_name="s")

    @functools.partial(
        pl.kernel,
        out_type=(
            jax.ShapeDtypeStruct((CB * S, HID), jnp.float32),
            jax.ShapeDtypeStruct((CB * PPAD, HID), jnp.float32),
        ),
        mesh=mesh,
        scratch_types=[
            pltpu.VMEM((ROWS_PER_W,), jnp.int32),
            pltpu.VMEM((2, CHUNK, HID), jnp.float32),
            pltpu.VMEM((PPAD,), jnp.int32),
            pltpu.VMEM((PPAD, HID), jnp.float32),
            pltpu.SemaphoreType.DMA,
            pltpu.SemaphoreType.DMA,
            pltpu.SemaphoreType.DMA,
        ],
    )
    def k(word_hbm, pemb_hbm, wids_hbm, pids_hbm, out_hbm, pout_hbm,
          idx_v, rows_v, pidx_v, prows_v, sem0, sem1, psem):
        sems = (sem0, sem1)
        w = lax.axis_index("s") * 2 + lax.axis_index("c")
        base = w * ROWS_PER_W
        pltpu.sync_copy(wids_hbm.at[pl.ds(base, ROWS_PER_W)], idx_v)
        pbase = (w // WPB) * PPAD
        pltpu.sync_copy(pids_hbm.at[pl.ds(pbase, PPAD)], pidx_v)
        pc = pltpu.make_async_copy(pemb_hbm.at[pidx_v], prows_v, psem)
        pc.start()

        copies = [None, None]
        copies[0] = pltpu.make_async_copy(
            word_hbm.at[idx_v.at[pl.ds(0, CHUNK)]], rows_v.at[0], sems[0])
        copies[0].start()
        for c in range(NCHUNK):
            buf = c % 2
            if c + 1 < NCHUNK:
                nbuf = (c + 1) % 2
                copies[nbuf] = pltpu.make_async_copy(
                    word_hbm.at[idx_v.at[pl.ds((c + 1) * CHUNK, CHUNK)]],
                    rows_v.at[nbuf], sems[nbuf])
                copies[nbuf].start()
            copies[buf].wait()
            pltpu.sync_copy(rows_v.at[buf],
                            out_hbm.at[pl.ds(base + c * CHUNK, CHUNK)])

        pc.wait()
        pltpu.sync_copy(prows_v, pout_hbm.at[pl.ds(pbase, PPAD)])

    return k(word_emb, prompt_emb, wids_c, pids_c)


def _tc_ln_body_first(g_ref, pg_ref, pos_ref, type_ref, gamma_ref, beta_ref,
                      o_ref):
    g = g_ref[...]
    pg = jnp.pad(pg_ref[...][:PLEN], ((1, g.shape[0] - PLEN - 1), (0, 0)))
    row = lax.broadcasted_iota(jnp.int32, (g.shape[0], 1), 0)
    mask = (row >= 1) & (row <= PLEN)
    x = jnp.where(mask, pg, g) + pos_ref[...] + type_ref[...]
    mu = jnp.mean(x, axis=-1, keepdims=True)
    d = x - mu
    var = jnp.mean(d * d, axis=-1, keepdims=True)
    o_ref[...] = d * lax.rsqrt(var + EPS) * gamma_ref[...] + beta_ref[...]


def _tc_ln_chunk(cidx, g_c, pg_c, pos_emb, type_row, gamma, beta):
    grid = (CB,)
    data_specs = [
        pl.BlockSpec((S, HID), lambda b: (b, 0)),
        pl.BlockSpec((PPAD, HID), lambda b: (b, 0)),
        pl.BlockSpec((S, HID), lambda b: (0, 0)),
        pl.BlockSpec((1, HID), lambda b: (0, 0)),
        pl.BlockSpec((1, HID), lambda b: (0, 0)),
        pl.BlockSpec((1, HID), lambda b: (0, 0)),
    ]
    out_spec = pl.BlockSpec((S, HID), lambda b: (cidx * CB + b, 0))
    out_shape = jax.ShapeDtypeStruct((B * S, HID), jnp.float32)
    return pl.pallas_call(
        _tc_ln_body_first,
        grid=grid,
        in_specs=data_specs,
        out_specs=out_spec,
        out_shape=out_shape,
    )(g_c, pg_c, pos_emb, type_row, gamma, beta)


def kernel(input_ids, prompt_ids, word_emb, prompt_emb, token_type_emb,
           pos_emb, ln_gamma, ln_beta):
    wids = input_ids.reshape(-1)
    pids = jnp.pad(prompt_ids, ((0, 0), (0, PPAD - PLEN)))
    type_row = token_type_emb[:1]
    gamma = ln_gamma.reshape(1, HID)
    beta = ln_beta.reshape(1, HID)
    out = None
    for c in range(NCHB):
        g_c, pg_c = _sc_gather_chunk(
            word_emb, prompt_emb,
            lax.slice(wids, (c * CB * S,), ((c + 1) * CB * S,)),
            pids[c * CB:(c + 1) * CB].reshape(-1))
        out = _tc_ln_chunk(c, g_c, pg_c, pos_emb, type_row, gamma, beta)
    return out.reshape(B, S, HID)
